# baseline (speedup 1.0000x reference)
import functools

import jax
import jax.numpy as jnp
from jax import lax
from jax.experimental import pallas as pl
from jax.experimental.pallas import tpu as pltpu
from jax.experimental.pallas import tpu_sc as plsc

_INFO = plsc.get_sparse_core_info()
_NC, _NS = _INFO.num_cores, _INFO.num_subcores
_NW = _NC * _NS


def kernel(new_k, new_v):
    shape = new_k.shape
    k2 = new_k.reshape(-1)
    v2 = new_v.reshape(-1)
    rows = k2.shape[0]
    rows_per = rows // _NW

    mesh = plsc.VectorSubcoreMesh(core_axis_name="c", subcore_axis_name="s")

    @functools.partial(
        pl.kernel,
        mesh=mesh,
        out_type=[
            jax.ShapeDtypeStruct(k2.shape, k2.dtype),
            jax.ShapeDtypeStruct(v2.shape, v2.dtype),
        ],
        scratch_types=[pltpu.SemaphoreType.DMA],
    )
    def _sc_copy(k_hbm, v_hbm, k_out, v_out, sem):
        wid = lax.axis_index("s") * _NC + lax.axis_index("c")
        base = wid * rows_per
        sl = pl.ds(base, rows_per)
        ck = pltpu.make_async_copy(k_hbm.at[sl], k_out.at[sl], sem)
        cv = pltpu.make_async_copy(v_hbm.at[sl], v_out.at[sl], sem)
        ck.start()
        cv.start()
        ck.wait()
        cv.wait()

    out = _sc_copy(k2, v2)
    return (out[0].reshape(shape), out[1].reshape(shape))


# speedup vs baseline: 38.7644x; 38.7644x over previous
import functools

import jax
import jax.numpy as jnp
from jax import lax
from jax.experimental import pallas as pl
from jax.experimental.pallas import tpu as pltpu
from jax.experimental.pallas import tpu_sc as plsc

_INFO = plsc.get_sparse_core_info()
_NC, _NS = _INFO.num_cores, _INFO.num_subcores
_NW = _NC * _NS

_CHUNK = 256


def kernel(new_k, new_v):
    shape = new_k.shape
    k2 = new_k.reshape(-1, shape[-1])
    v2 = new_v.reshape(-1, shape[-1])
    rows, cols = k2.shape
    rows_per = rows // _NW
    nch = rows_per // _CHUNK

    mesh = plsc.VectorSubcoreMesh(core_axis_name="c", subcore_axis_name="s")

    @functools.partial(
        pl.kernel,
        mesh=mesh,
        out_type=[
            jax.ShapeDtypeStruct(k2.shape, k2.dtype),
            jax.ShapeDtypeStruct(v2.shape, v2.dtype),
        ],
        scratch_types=[
            pltpu.VMEM((2, _CHUNK, 128), jnp.float32),
            pltpu.SemaphoreType.DMA((2,)),
            pltpu.SemaphoreType.DMA((2,)),
        ],
    )
    def _sc_copy(k_hbm, v_hbm, k_out, v_out, bufs, sin, sout):
        wid = lax.axis_index("s") * _NC + lax.axis_index("c")
        base = wid * rows_per

        seq = [(k_hbm, k_out, i) for i in range(nch)]
        seq += [(v_hbm, v_out, i) for i in range(nch)]

        def in_copy(t):
            src, _, i = seq[t]
            b = t % 2
            sl = pl.ds(base + i * _CHUNK, _CHUNK)
            return pltpu.make_async_copy(src.at[sl], bufs.at[b], sin.at[b])

        def out_copy(t):
            _, dst, i = seq[t]
            b = t % 2
            sl = pl.ds(base + i * _CHUNK, _CHUNK)
            return pltpu.make_async_copy(bufs.at[b], dst.at[sl], sout.at[b])

        in_copy(0).start()
        for t in range(len(seq)):
            in_copy(t).wait()
            if t + 1 < len(seq):
                in_copy(t + 1).start()
            oc = out_copy(t)
            oc.start()
            oc.wait()

    out = _sc_copy(k2, v2)
    return (out[0].reshape(shape), out[1].reshape(shape))


# speedup vs baseline: 42.5687x; 1.0981x over previous
import functools

import jax
import jax.numpy as jnp
from jax import lax
from jax.experimental import pallas as pl
from jax.experimental.pallas import tpu as pltpu
from jax.experimental.pallas import tpu_sc as plsc

_INFO = plsc.get_sparse_core_info()
_NC, _NS = _INFO.num_cores, _INFO.num_subcores
_NW = _NC * _NS

_TC_BLOCK_ROWS = 8192
_SC_CHUNK = 256


def _tc_copy_body(x_in, x_out):
    x_out[...] = x_in[...]


def _tc_copy(x2):
    rows, cols = x2.shape
    spec = pl.BlockSpec((_TC_BLOCK_ROWS, cols), lambda i: (i, 0))
    return pl.pallas_call(
        _tc_copy_body,
        grid=(rows // _TC_BLOCK_ROWS,),
        in_specs=[spec],
        out_specs=spec,
        out_shape=jax.ShapeDtypeStruct(x2.shape, x2.dtype),
        compiler_params=pltpu.CompilerParams(
            dimension_semantics=("parallel",),
            skip_device_barrier=True,
            disable_bounds_checks=True,
        ),
    )(x2)


def _sc_copy(x2):
    rows, cols = x2.shape
    rows_per = rows // _NW
    nch = rows_per // _SC_CHUNK

    mesh = plsc.VectorSubcoreMesh(core_axis_name="c", subcore_axis_name="s")

    @functools.partial(
        pl.kernel,
        mesh=mesh,
        out_type=jax.ShapeDtypeStruct(x2.shape, x2.dtype),
        scratch_types=[
            pltpu.VMEM((2, _SC_CHUNK, 128), jnp.float32),
            pltpu.SemaphoreType.DMA((2,)),
            pltpu.SemaphoreType.DMA((2,)),
        ],
    )
    def _body(x_hbm, x_out, bufs, sin, sout):
        wid = lax.axis_index("s") * _NC + lax.axis_index("c")
        base = wid * rows_per

        def in_copy(t):
            b = t % 2
            sl = pl.ds(base + t * _SC_CHUNK, _SC_CHUNK)
            return pltpu.make_async_copy(x_hbm.at[sl], bufs.at[b], sin.at[b])

        def out_copy(t):
            b = t % 2
            sl = pl.ds(base + t * _SC_CHUNK, _SC_CHUNK)
            return pltpu.make_async_copy(bufs.at[b], x_out.at[sl], sout.at[b])

        in_copy(0).start()
        for t in range(nch):
            in_copy(t).wait()
            if t + 1 < nch:
                in_copy(t + 1).start()
            oc = out_copy(t)
            oc.start()
            oc.wait()

    return _body(x2)


def kernel(new_k, new_v):
    shape = new_k.shape
    k2 = new_k.reshape(-1, shape[-1])
    v2 = new_v.reshape(-1, shape[-1])
    out_k = _tc_copy(k2)
    out_v = _sc_copy(v2)
    return (out_k.reshape(shape), out_v.reshape(shape))
